# baseline (speedup 1.0000x reference)
import jax
import jax.numpy as jnp
from jax.experimental import pallas as pl
from jax.experimental.pallas import tpu as pltpu

N_BLK = 2048


def _matmul_kernel(q_ref, n_ref, o_ref):
    q = q_ref[...]
    n = n_ref[...]
    o_ref[...] = jax.lax.dot_general(
        q, n, (((1,), (1,)), ((), ())),
        preferred_element_type=jnp.float32)


def kernel(queries_embeddings, nodes_embeddings, k):
    Q, D = queries_embeddings.shape
    N, _ = nodes_embeddings.shape
    nblocks = pl.cdiv(N, N_BLK)
    scores = pl.pallas_call(
        _matmul_kernel,
        grid=(nblocks,),
        in_specs=[
            pl.BlockSpec((Q, D), lambda i: (0, 0)),
            pl.BlockSpec((N_BLK, D), lambda i: (i, 0)),
        ],
        out_specs=pl.BlockSpec((Q, N_BLK), lambda i: (0, i)),
        out_shape=jax.ShapeDtypeStruct((Q, N), jnp.float32),
        compiler_params=pltpu.CompilerParams(
            dimension_semantics=("parallel",)),
    )(queries_embeddings, nodes_embeddings)
    top_scores, childs = jax.lax.top_k(scores, min(100, N))
    return (childs, top_scores)


# speedup vs baseline: 1.2854x; 1.2854x over previous
import functools

import jax
import jax.numpy as jnp
from jax.experimental import pallas as pl
from jax.experimental.pallas import tpu as pltpu

N_BLK = 2048
Q_BLK = 512
CH = 128
NEG = float(jnp.finfo(jnp.float32).min)


def _partner(x, j):
    lane = jax.lax.broadcasted_iota(jnp.int32, (1, CH), 1)
    bit = (lane & j) != 0
    return jnp.where(bit, jnp.roll(x, j, axis=-1), jnp.roll(x, -j, axis=-1))


def _stage(v, i, j, up_xor_islower):
    pv = _partner(v, j)
    pi = _partner(i, j)
    self_first = (v > pv) | ((v == pv) & (i < pi))
    take = self_first ^ up_xor_islower
    return jnp.where(take, v, pv), jnp.where(take, i, pi)


def _lane_consts(j, k, descending):
    lane = jax.lax.broadcasted_iota(jnp.int32, (1, CH), 1)
    up = (lane & k) == 0
    if descending:
        up = ~up
    is_lower = (lane & j) == 0
    return is_lower ^ (~up)


def _sort128(v, i, descending):
    k = 2
    while k <= CH:
        j = k // 2
        while j >= 1:
            v, i = _stage(v, i, j, _lane_consts(j, k, descending))
            j //= 2
        k *= 2
    return v, i


def _clean128_desc(v, i):
    j = CH // 2
    while j >= 1:
        v, i = _stage(v, i, j, _lane_consts(j, CH, True))
        j //= 2
    return v, i


def _merge_into(buf_v, buf_i, v_asc, i_asc):
    self_first = (buf_v > v_asc) | ((buf_v == v_asc) & (buf_i < i_asc))
    w_v = jnp.where(self_first, buf_v, v_asc)
    w_i = jnp.where(self_first, buf_i, i_asc)
    return _clean128_desc(w_v, w_i)


def _topk_kernel(n_total, q_ref, n_ref, childs_ref, scores_ref, bv_ref, bi_ref):
    nb = pl.program_id(1)
    nblocks = pl.num_programs(1)

    @pl.when(nb == 0)
    def _init():
        bv_ref[...] = jnp.full_like(bv_ref, NEG)
        bi_ref[...] = jnp.zeros_like(bi_ref)

    q = q_ref[...]
    n = n_ref[...]
    s = jax.lax.dot_general(q, n, (((1,), (1,)), ((), ())),
                            preferred_element_type=jnp.float32)
    lane = jax.lax.broadcasted_iota(jnp.int32, (1, CH), 1)
    col0 = nb * N_BLK

    bv, bi = bv_ref[...], bi_ref[...]
    for j in range(N_BLK // CH):
        v = s[:, j * CH:(j + 1) * CH]
        cols = col0 + j * CH + lane
        v = jnp.where(cols < n_total, v, NEG)
        ci = jnp.broadcast_to(cols, (Q_BLK, CH))
        v, ci = _sort128(v, ci, descending=False)
        bv, bi = _merge_into(bv, bi, v, ci)
    bv_ref[...] = bv
    bi_ref[...] = bi

    @pl.when(nb == nblocks - 1)
    def _emit():
        childs_ref[...] = bi[:, :100]
        scores_ref[...] = bv[:, :100]


def kernel(queries_embeddings, nodes_embeddings, k):
    Q, D = queries_embeddings.shape
    N, _ = nodes_embeddings.shape
    kk = min(100, N)
    nblocks = pl.cdiv(N, N_BLK)
    childs, scores = pl.pallas_call(
        functools.partial(_topk_kernel, N),
        grid=(Q // Q_BLK, nblocks),
        in_specs=[
            pl.BlockSpec((Q_BLK, D), lambda r, i: (r, 0)),
            pl.BlockSpec((N_BLK, D), lambda r, i: (i, 0)),
        ],
        out_specs=[
            pl.BlockSpec((Q_BLK, kk), lambda r, i: (r, 0)),
            pl.BlockSpec((Q_BLK, kk), lambda r, i: (r, 0)),
        ],
        out_shape=[
            jax.ShapeDtypeStruct((Q, kk), jnp.int32),
            jax.ShapeDtypeStruct((Q, kk), jnp.float32),
        ],
        scratch_shapes=[
            pltpu.VMEM((Q_BLK, CH), jnp.float32),
            pltpu.VMEM((Q_BLK, CH), jnp.int32),
        ],
        compiler_params=pltpu.CompilerParams(
            dimension_semantics=("parallel", "arbitrary")),
    )(queries_embeddings, nodes_embeddings)
    return (childs, scores)


# speedup vs baseline: 7.4023x; 5.7586x over previous
import functools

import jax
import jax.numpy as jnp
from jax import lax
from jax.experimental import pallas as pl
from jax.experimental.pallas import tpu as pltpu
from jax.experimental.pallas import tpu_sc as plsc

N_BLK = 2048
Q_BLK = 512
CH = 128
NEG = float(jnp.finfo(jnp.float32).min)
H = 100
NW = 32
GATHER_CHUNK = 640


def _partner(x, j):
    lane = jax.lax.broadcasted_iota(jnp.int32, (1, CH), 1)
    bit = (lane & j) != 0
    return jnp.where(bit, jnp.roll(x, j, axis=-1), jnp.roll(x, -j, axis=-1))


def _stage(v, i, j, take_const):
    pv = _partner(v, j)
    pi = _partner(i, j)
    self_first = (v > pv) | ((v == pv) & (i < pi))
    take = self_first ^ take_const
    return jnp.where(take, v, pv), jnp.where(take, i, pi)


def _lane_consts(j, k, descending):
    lane = jax.lax.broadcasted_iota(jnp.int32, (1, CH), 1)
    up = (lane & k) == 0
    if descending:
        up = ~up
    is_lower = (lane & j) == 0
    return is_lower ^ (~up)


def _sort128(v, i, descending):
    k = 2
    while k <= CH:
        j = k // 2
        while j >= 1:
            v, i = _stage(v, i, j, _lane_consts(j, k, descending))
            j //= 2
        k *= 2
    return v, i


def _clean128_desc(v, i):
    j = CH // 2
    while j >= 1:
        v, i = _stage(v, i, j, _lane_consts(j, CH, True))
        j //= 2
    return v, i


def _merge_into(buf_v, buf_i, v_asc, i_asc):
    self_first = (buf_v > v_asc) | ((buf_v == v_asc) & (buf_i < i_asc))
    w_v = jnp.where(self_first, buf_v, v_asc)
    w_i = jnp.where(self_first, buf_i, i_asc)
    return _clean128_desc(w_v, w_i)


def _kernel_a(n_total, q_ref, n_ref, s_ref, gmax_ref):
    nb = pl.program_id(1)
    q = q_ref[...]
    n = n_ref[...]
    s = jax.lax.dot_general(q, n, (((1,), (1,)), ((), ())),
                            preferred_element_type=jnp.float32)
    lane = jax.lax.broadcasted_iota(jnp.int32, (1, CH), 1)
    col0 = nb * N_BLK
    masked = []
    gmaxes = []
    for j in range(N_BLK // CH):
        v = s[:, j * CH:(j + 1) * CH]
        cols = col0 + j * CH + lane
        v = jnp.where(cols < n_total, v, NEG)
        masked.append(v)
        gmaxes.append(jnp.max(v, axis=1, keepdims=True))
    s_ref[...] = jnp.concatenate(masked, axis=1)
    gmax_ref[...] = jnp.concatenate(gmaxes, axis=1)[:, None, None, :]


def _kernel_b(gmax_ref, ids_ref):
    gm = gmax_ref[...]
    rows = gm.shape[0]
    ngroups = gm.shape[1]
    npad = (-ngroups) % CH
    if npad:
        gm = jnp.concatenate(
            [gm, jnp.full((rows, npad), NEG, jnp.float32)], axis=1)
    lane = jax.lax.broadcasted_iota(jnp.int32, (1, CH), 1)
    bv = bi = None
    for j in range((ngroups + npad) // CH):
        v = gm[:, j * CH:(j + 1) * CH]
        gid = jnp.broadcast_to(j * CH + lane, (rows, CH))
        if bv is None:
            bv, bi = _sort128(v, gid, descending=True)
        else:
            v, gid = _sort128(v, gid, descending=False)
            bv, bi = _merge_into(bv, bi, v, gid)
    row_global = (pl.program_id(0) * Q_BLK
                  + jax.lax.broadcasted_iota(jnp.int32, (rows, 1), 0))
    flat = row_global * ngroups + bi
    ids_ref[...] = flat[:, :H]


def _sc_gather(table, ids_c, n_rows_out):
    nsteps = (n_rows_out // NW) // GATHER_CHUNK

    @functools.partial(
        pl.kernel,
        out_type=jax.ShapeDtypeStruct((n_rows_out, CH), jnp.float32),
        mesh=plsc.VectorSubcoreMesh(core_axis_name="c", subcore_axis_name="s"),
        scratch_types=[
            pltpu.VMEM((n_rows_out // NW,), jnp.int32),
            pltpu.VMEM((GATHER_CHUNK, CH), jnp.float32),
            pltpu.SemaphoreType.DMA,
        ],
    )
    def k(table_hbm, ids_hbm, out_hbm, idx_v, rows_v, sem):
        wid = lax.axis_index("s") * 2 + lax.axis_index("c")
        pltpu.sync_copy(ids_hbm.at[wid], idx_v)
        base = wid * (n_rows_out // NW)
        for c in range(nsteps):
            pltpu.async_copy(
                table_hbm.at[idx_v.at[pl.ds(c * GATHER_CHUNK, GATHER_CHUNK)]],
                rows_v, sem).wait()
            pltpu.sync_copy(
                rows_v, out_hbm.at[pl.ds(base + c * GATHER_CHUNK,
                                         GATHER_CHUNK)])

    return k(table, ids_c)


def _kernel_c(ngroups, g_ref, ids_ref, childs_ref, scores_ref, bv_ref, bi_ref):
    h = pl.program_id(1)

    @pl.when(h == 0)
    def _init():
        bv_ref[...] = jnp.full_like(bv_ref, NEG)
        bi_ref[...] = jnp.zeros_like(bi_ref)

    v = g_ref[...][0]
    rows = v.shape[0]
    lane = jax.lax.broadcasted_iota(jnp.int32, (1, CH), 1)
    row_global = (pl.program_id(0) * Q_BLK
                  + jax.lax.broadcasted_iota(jnp.int32, (rows, 1), 0))
    ids_all = ids_ref[...]
    lane_h = jax.lax.broadcasted_iota(jnp.int32, (1, H), 1)
    flat = jnp.sum(jnp.where(lane_h == h, ids_all, 0), axis=1, keepdims=True)
    g = flat - row_global * ngroups
    cols = g * CH + lane
    v, ci = _sort128(v, cols, descending=False)
    bv, bi = _merge_into(bv_ref[...], bi_ref[...], v, ci)
    bv_ref[...] = bv
    bi_ref[...] = bi

    @pl.when(h == pl.num_programs(1) - 1)
    def _emit():
        childs_ref[...] = bi[:, :H]
        scores_ref[...] = bv[:, :H]


def kernel(queries_embeddings, nodes_embeddings, k):
    Q, D = queries_embeddings.shape
    N, _ = nodes_embeddings.shape
    nblocks = pl.cdiv(N, N_BLK)
    ngroups = nblocks * (N_BLK // CH)
    npad_cols = ngroups * CH

    scores, gmax4 = pl.pallas_call(
        functools.partial(_kernel_a, N),
        grid=(Q // Q_BLK, nblocks),
        in_specs=[
            pl.BlockSpec((Q_BLK, D), lambda r, i: (r, 0)),
            pl.BlockSpec((N_BLK, D), lambda r, i: (i, 0)),
        ],
        out_specs=[
            pl.BlockSpec((Q_BLK, N_BLK), lambda r, i: (r, i)),
            pl.BlockSpec((Q_BLK, 1, 1, N_BLK // CH), lambda r, i: (r, i, 0, 0)),
        ],
        out_shape=[
            jax.ShapeDtypeStruct((Q, npad_cols), jnp.float32),
            jax.ShapeDtypeStruct((Q, nblocks, 1, N_BLK // CH), jnp.float32),
        ],
        compiler_params=pltpu.CompilerParams(
            dimension_semantics=("parallel", "arbitrary")),
    )(queries_embeddings, nodes_embeddings)

    gmax = gmax4.reshape(Q, ngroups)
    ids = pl.pallas_call(
        _kernel_b,
        grid=(Q // Q_BLK,),
        in_specs=[pl.BlockSpec((Q_BLK, ngroups), lambda r: (r, 0))],
        out_specs=pl.BlockSpec((Q_BLK, H), lambda r: (r, 0)),
        out_shape=jax.ShapeDtypeStruct((Q, H), jnp.int32),
        compiler_params=pltpu.CompilerParams(
            dimension_semantics=("parallel",)),
    )(gmax)

    rpw = Q // NW
    ids_c = ids.reshape(NW, rpw, H).transpose(0, 2, 1).reshape(NW, H * rpw)
    table = scores.reshape(Q * ngroups, CH)
    gathered = _sc_gather(table, ids_c, Q * H)
    g4 = (gathered.reshape(NW, H, rpw, CH)
          .transpose(1, 0, 2, 3).reshape(H, Q, CH))

    childs, out_scores = pl.pallas_call(
        functools.partial(_kernel_c, ngroups),
        grid=(Q // Q_BLK, H),
        in_specs=[
            pl.BlockSpec((1, Q_BLK, CH), lambda r, h: (h, r, 0)),
            pl.BlockSpec((Q_BLK, H), lambda r, h: (r, 0)),
        ],
        out_specs=[
            pl.BlockSpec((Q_BLK, H), lambda r, h: (r, 0)),
            pl.BlockSpec((Q_BLK, H), lambda r, h: (r, 0)),
        ],
        out_shape=[
            jax.ShapeDtypeStruct((Q, H), jnp.int32),
            jax.ShapeDtypeStruct((Q, H), jnp.float32),
        ],
        scratch_shapes=[
            pltpu.VMEM((Q_BLK, CH), jnp.float32),
            pltpu.VMEM((Q_BLK, CH), jnp.int32),
        ],
        compiler_params=pltpu.CompilerParams(
            dimension_semantics=("parallel", "arbitrary")),
    )(g4, ids)
    return (childs, out_scores)


# speedup vs baseline: 11.1650x; 1.5083x over previous
import functools

import jax
import jax.numpy as jnp
from jax import lax
from jax.experimental import pallas as pl
from jax.experimental.pallas import tpu as pltpu
from jax.experimental.pallas import tpu_sc as plsc

N_BLK = 2048
Q_BLK = 512
CH = 128
NEG = float(jnp.finfo(jnp.float32).min)
H = 100
NW = 32
GATHER_CHUNK = 640


def _partner(x, j):
    lane = jax.lax.broadcasted_iota(jnp.int32, x.shape, x.ndim - 1)
    return jnp.take_along_axis(x, lane ^ j, axis=-1)


def _stage(v, i, j, take_const):
    pv = _partner(v, j)
    pi = _partner(i, j)
    self_first = (v > pv) | ((v == pv) & (i < pi))
    take = self_first ^ take_const
    return jnp.where(take, v, pv), jnp.where(take, i, pi)


def _lane_consts(j, k, descending):
    lane = jax.lax.broadcasted_iota(jnp.int32, (1, CH), 1)
    up = (lane & k) == 0
    if descending:
        up = ~up
    is_lower = (lane & j) == 0
    return is_lower ^ (~up)


def _sort128(v, i, descending):
    k = 2
    while k <= CH:
        j = k // 2
        while j >= 1:
            v, i = _stage(v, i, j, _lane_consts(j, k, descending))
            j //= 2
        k *= 2
    return v, i


def _clean128_desc(v, i):
    j = CH // 2
    while j >= 1:
        v, i = _stage(v, i, j, _lane_consts(j, CH, True))
        j //= 2
    return v, i


def _merge_into(buf_v, buf_i, v_asc, i_asc):
    self_first = (buf_v > v_asc) | ((buf_v == v_asc) & (buf_i < i_asc))
    w_v = jnp.where(self_first, buf_v, v_asc)
    w_i = jnp.where(self_first, buf_i, i_asc)
    return _clean128_desc(w_v, w_i)


def _kernel_a(n_total, q_ref, n_ref, s_ref, gmax_ref):
    nb = pl.program_id(1)
    q = q_ref[...]
    n = n_ref[...]
    s = jax.lax.dot_general(q, n, (((1,), (1,)), ((), ())),
                            preferred_element_type=jnp.float32)
    lane = jax.lax.broadcasted_iota(jnp.int32, (1, CH), 1)
    col0 = nb * N_BLK
    masked = []
    gmaxes = []
    for j in range(N_BLK // CH):
        v = s[:, j * CH:(j + 1) * CH]
        cols = col0 + j * CH + lane
        v = jnp.where(cols < n_total, v, NEG)
        masked.append(v)
        gmaxes.append(jnp.max(v, axis=1, keepdims=True))
    s_ref[...] = jnp.concatenate(masked, axis=1)
    gmax_ref[...] = jnp.concatenate(gmaxes, axis=1)[:, None, None, :]


def _kernel_b(gmax_ref, ids_ref):
    gm = gmax_ref[...]
    rows = gm.shape[0]
    ngroups = gm.shape[1]
    npad = (-ngroups) % CH
    if npad:
        gm = jnp.concatenate(
            [gm, jnp.full((rows, npad), NEG, jnp.float32)], axis=1)
    lane = jax.lax.broadcasted_iota(jnp.int32, (1, CH), 1)
    bv = bi = None
    for j in range((ngroups + npad) // CH):
        v = gm[:, j * CH:(j + 1) * CH]
        gid = jnp.broadcast_to(j * CH + lane, (rows, CH))
        if bv is None:
            bv, bi = _sort128(v, gid, descending=True)
        else:
            v, gid = _sort128(v, gid, descending=False)
            bv, bi = _merge_into(bv, bi, v, gid)
    row_global = (pl.program_id(0) * Q_BLK
                  + jax.lax.broadcasted_iota(jnp.int32, (rows, 1), 0))
    flat = row_global * ngroups + bi
    ids_ref[...] = flat[:, :H]


def _sc_gather(table, ids_c, n_rows_out):
    nsteps = (n_rows_out // NW) // GATHER_CHUNK

    @functools.partial(
        pl.kernel,
        out_type=jax.ShapeDtypeStruct((n_rows_out, CH), jnp.float32),
        mesh=plsc.VectorSubcoreMesh(core_axis_name="c", subcore_axis_name="s"),
        scratch_types=[
            pltpu.VMEM((n_rows_out // NW,), jnp.int32),
            pltpu.VMEM((GATHER_CHUNK, CH), jnp.float32),
            pltpu.SemaphoreType.DMA,
        ],
    )
    def k(table_hbm, ids_hbm, out_hbm, idx_v, rows_v, sem):
        wid = lax.axis_index("s") * 2 + lax.axis_index("c")
        pltpu.sync_copy(ids_hbm.at[wid], idx_v)
        base = wid * (n_rows_out // NW)
        for c in range(nsteps):
            pltpu.async_copy(
                table_hbm.at[idx_v.at[pl.ds(c * GATHER_CHUNK, GATHER_CHUNK)]],
                rows_v, sem).wait()
            pltpu.sync_copy(
                rows_v, out_hbm.at[pl.ds(base + c * GATHER_CHUNK,
                                         GATHER_CHUNK)])

    return k(table, ids_c)


def _kernel_c(ngroups, g_ref, ids_ref, childs_ref, scores_ref, bv_ref, bi_ref):
    h = pl.program_id(1)

    @pl.when(h == 0)
    def _init():
        bv_ref[...] = jnp.full_like(bv_ref, NEG)
        bi_ref[...] = jnp.zeros_like(bi_ref)

    v = g_ref[...][0]
    rows = v.shape[0]
    lane = jax.lax.broadcasted_iota(jnp.int32, (1, CH), 1)
    row_global = (pl.program_id(0) * Q_BLK
                  + jax.lax.broadcasted_iota(jnp.int32, (rows, 1), 0))
    ids_all = ids_ref[...]
    lane_h = jax.lax.broadcasted_iota(jnp.int32, (1, H), 1)
    flat = jnp.sum(jnp.where(lane_h == h, ids_all, 0), axis=1, keepdims=True)
    g = flat - row_global * ngroups
    cols = g * CH + lane
    v, ci = _sort128(v, cols, descending=False)
    bv, bi = _merge_into(bv_ref[...], bi_ref[...], v, ci)
    bv_ref[...] = bv
    bi_ref[...] = bi

    @pl.when(h == pl.num_programs(1) - 1)
    def _emit():
        childs_ref[...] = bi[:, :H]
        scores_ref[...] = bv[:, :H]


def kernel(queries_embeddings, nodes_embeddings, k):
    Q, D = queries_embeddings.shape
    N, _ = nodes_embeddings.shape
    nblocks = pl.cdiv(N, N_BLK)
    ngroups = nblocks * (N_BLK // CH)
    npad_cols = ngroups * CH

    scores, gmax4 = pl.pallas_call(
        functools.partial(_kernel_a, N),
        grid=(Q // Q_BLK, nblocks),
        in_specs=[
            pl.BlockSpec((Q_BLK, D), lambda r, i: (r, 0)),
            pl.BlockSpec((N_BLK, D), lambda r, i: (i, 0)),
        ],
        out_specs=[
            pl.BlockSpec((Q_BLK, N_BLK), lambda r, i: (r, i)),
            pl.BlockSpec((Q_BLK, 1, 1, N_BLK // CH), lambda r, i: (r, i, 0, 0)),
        ],
        out_shape=[
            jax.ShapeDtypeStruct((Q, npad_cols), jnp.float32),
            jax.ShapeDtypeStruct((Q, nblocks, 1, N_BLK // CH), jnp.float32),
        ],
        compiler_params=pltpu.CompilerParams(
            dimension_semantics=("parallel", "arbitrary")),
    )(queries_embeddings, nodes_embeddings)

    gmax = gmax4.reshape(Q, ngroups)
    ids = pl.pallas_call(
        _kernel_b,
        grid=(Q // Q_BLK,),
        in_specs=[pl.BlockSpec((Q_BLK, ngroups), lambda r: (r, 0))],
        out_specs=pl.BlockSpec((Q_BLK, H), lambda r: (r, 0)),
        out_shape=jax.ShapeDtypeStruct((Q, H), jnp.int32),
        compiler_params=pltpu.CompilerParams(
            dimension_semantics=("parallel",)),
    )(gmax)

    rpw = Q // NW
    ids_c = ids.reshape(NW, rpw, H).transpose(0, 2, 1).reshape(NW, H * rpw)
    table = scores.reshape(Q * ngroups, CH)
    gathered = _sc_gather(table, ids_c, Q * H)
    g4 = (gathered.reshape(NW, H, rpw, CH)
          .transpose(1, 0, 2, 3).reshape(H, Q, CH))

    childs, out_scores = pl.pallas_call(
        functools.partial(_kernel_c, ngroups),
        grid=(Q // Q_BLK, H),
        in_specs=[
            pl.BlockSpec((1, Q_BLK, CH), lambda r, h: (h, r, 0)),
            pl.BlockSpec((Q_BLK, H), lambda r, h: (r, 0)),
        ],
        out_specs=[
            pl.BlockSpec((Q_BLK, H), lambda r, h: (r, 0)),
            pl.BlockSpec((Q_BLK, H), lambda r, h: (r, 0)),
        ],
        out_shape=[
            jax.ShapeDtypeStruct((Q, H), jnp.int32),
            jax.ShapeDtypeStruct((Q, H), jnp.float32),
        ],
        scratch_shapes=[
            pltpu.VMEM((Q_BLK, CH), jnp.float32),
            pltpu.VMEM((Q_BLK, CH), jnp.int32),
        ],
        compiler_params=pltpu.CompilerParams(
            dimension_semantics=("parallel", "arbitrary")),
    )(g4, ids)
    return (childs, out_scores)
